# baseline (speedup 1.0000x reference)
import functools

import jax
import jax.numpy as jnp
from jax import lax
from jax.experimental import pallas as pl
from jax.experimental.pallas import tpu as pltpu
from jax.experimental.pallas import tpu_sc as plsc

NC = 2
NS = 16
LANES = 16
NW = NC * NS

HCB = 8
NSETS = 4
GATHER_ROWS = 80


def _make_sc_scores(n_items, ctx, d, ipw):
    n_half = ipw // HCB
    nj = d // LANES
    items_per_buf = GATHER_ROWS // ctx
    mesh = plsc.VectorSubcoreMesh(core_axis_name="c", subcore_axis_name="s")

    @functools.partial(
        pl.kernel,
        mesh=mesh,
        compiler_params=pltpu.CompilerParams(needs_layout_passes=False),
        out_type=jax.ShapeDtypeStruct((n_items,), jnp.float32),
        scratch_types=[
            pltpu.VMEM((ipw * ctx,), jnp.int32),
            pltpu.VMEM((ipw,), jnp.int32),
            pltpu.VMEM((GATHER_ROWS, d), jnp.float32),
            pltpu.VMEM((GATHER_ROWS, d), jnp.float32),
            pltpu.VMEM((GATHER_ROWS, d), jnp.float32),
            pltpu.VMEM((GATHER_ROWS, d), jnp.float32),
            pltpu.VMEM((GATHER_ROWS, d), jnp.float32),
            pltpu.VMEM((GATHER_ROWS, d), jnp.float32),
            pltpu.VMEM((GATHER_ROWS, d), jnp.float32),
            pltpu.VMEM((GATHER_ROWS, d), jnp.float32),
            pltpu.VMEM((HCB, d), jnp.float32),
            pltpu.VMEM((HCB, d), jnp.float32),
            pltpu.VMEM((HCB, d), jnp.float32),
            pltpu.VMEM((HCB, d), jnp.float32),
            pltpu.VMEM((HCB, d), jnp.float32),
            pltpu.VMEM((HCB, d), jnp.float32),
            pltpu.VMEM((HCB, d), jnp.float32),
            pltpu.VMEM((HCB, d), jnp.float32),
            pltpu.VMEM((HCB, d), jnp.float32),
            pltpu.VMEM((GATHER_ROWS,), jnp.int32),
            pltpu.VMEM((GATHER_ROWS,), jnp.int32),
            pltpu.VMEM((ipw,), jnp.float32),
            pltpu.VMEM((LANES, LANES), jnp.float32),
            pltpu.VMEM_SHARED((NS * HCB, d), jnp.float32),
            pltpu.VMEM_SHARED((NS * HCB, d), jnp.float32),
            pltpu.VMEM_SHARED((NS * HCB, d), jnp.float32),
            pltpu.VMEM_SHARED((NS * HCB, d), jnp.float32),
            pltpu.SemaphoreType.DMA,
            pltpu.SemaphoreType.DMA,
            pltpu.SemaphoreType.DMA,
            pltpu.SemaphoreType.DMA,
            pltpu.SemaphoreType.DMA,
            pltpu.SemaphoreType.DMA,
            pltpu.SemaphoreType.DMA,
            pltpu.SemaphoreType.DMA,
            pltpu.SemaphoreType.DMA,
            pltpu.SemaphoreType.DMA,
            pltpu.SemaphoreType.DMA,
            pltpu.SemaphoreType.DMA,
        ],
    )
    def sc_scores(uidx_hbm, vidx_hbm, ut_hbm, vt_hbm, out_hbm,
                  uidx_v, vidx_v,
                  r00, r01, r10, r11, r20, r21, r30, r31,
                  vr0, vr1, vr2, vr3,
                  pb0, pb1, pb2, pb3, zbuf,
                  pidx0, pidx1,
                  scores_v, pmat,
                  acc0, acc1, acc2, acc3,
                  gsem0, gsem1, gsem2, gsem3,
                  psem0, psem1, psem2, psem3,
                  csem0, csem1, csem2, csem3):
        wid = lax.axis_index("s") * NC + lax.axis_index("c")
        sid = lax.axis_index("s")
        rows_sets = ((r00, r01), (r10, r11), (r20, r21), (r30, r31))
        vr_sets = (vr0, vr1, vr2, vr3)
        pb_sets = (pb0, pb1, pb2, pb3)
        acc_sets = (acc0, acc1, acc2, acc3)
        poolidx = (pidx0, pidx1)
        gsems = (gsem0, gsem1, gsem2, gsem3)
        psems = (psem0, psem1, psem2, psem3)
        csems = (csem0, csem1, csem2, csem3)

        zeros16 = jnp.zeros((LANES,), jnp.float32)
        lanes = lax.iota(jnp.int32, LANES)
        my_rows = sid * HCB

        for g in range(2):
            for k in range(GATHER_ROWS // LANES):
                r = k * LANES + lanes
                poolidx[g][pl.ds(k * LANES, LANES)] = (
                    my_rows + g * items_per_buf + r // ctx)

        def zb_body(i, c):
            for q in range(nj):
                zbuf[i, pl.ds(LANES * q, LANES)] = zeros16
            return c
        lax.fori_loop(0, HCB, zb_body, 0)
        for a in acc_sets:
            pltpu.sync_copy(zbuf, a.at[pl.ds(my_rows, HCB)])

        pltpu.sync_copy(uidx_hbm.at[pl.ds(wid * (ipw * ctx), ipw * ctx)], uidx_v)
        pltpu.sync_copy(vidx_hbm.at[pl.ds(wid * ipw, ipw)], vidx_v)

        def fire_gather(h, s):
            rows, vr, sem = rows_sets[s], vr_sets[s], gsems[s]
            bu = h * (HCB * ctx)
            for g in range(2):
                pltpu.make_async_copy(
                    ut_hbm.at[uidx_v.at[pl.ds(bu + g * GATHER_ROWS, GATHER_ROWS)]],
                    rows[g], sem).start()
            pltpu.make_async_copy(
                vt_hbm.at[vidx_v.at[pl.ds(h * HCB, HCB)]], vr, sem).start()

        def drain_gather(h, s):
            rows, vr, sem = rows_sets[s], vr_sets[s], gsems[s]
            bu = h * (HCB * ctx)
            for g in range(2):
                pltpu.make_async_copy(
                    ut_hbm.at[uidx_v.at[pl.ds(bu + g * GATHER_ROWS, GATHER_ROWS)]],
                    rows[g], sem).wait()
            pltpu.make_async_copy(
                vt_hbm.at[vidx_v.at[pl.ds(h * HCB, HCB)]], vr, sem).wait()

        def fire_pool(s):
            rows, acc, sem = rows_sets[s], acc_sets[s], psems[s]
            for g in range(2):
                pltpu.async_copy(rows[g], acc.at[poolidx[g]], sem, add=True)

        def drain_pool(s):
            rows, acc, sem = rows_sets[s], acc_sets[s], psems[s]
            for g in range(2):
                pltpu.make_async_copy(rows[g], acc.at[poolidx[g]], sem).wait()

        def fire_copyback(s):
            pltpu.make_async_copy(
                acc_sets[s].at[pl.ds(my_rows, HCB)], pb_sets[s], csems[s]).start()

        def drain_copyback(s):
            pltpu.make_async_copy(
                acc_sets[s].at[pl.ds(my_rows, HCB)], pb_sets[s], csems[s]).wait()

        def fire_zero(s):
            pltpu.make_async_copy(
                zbuf, acc_sets[s].at[pl.ds(my_rows, HCB)], psems[s]).start()

        def drain_zero(s):
            pltpu.make_async_copy(
                zbuf, acc_sets[s].at[pl.ds(my_rows, HCB)], psems[s]).wait()

        def dot(s, prow_base):
            pb, vr = pb_sets[s], vr_sets[s]

            def item_body(i, carry):
                p = pb[i, pl.ds(0, LANES)] * vr[i, pl.ds(0, LANES)]
                for q in range(1, nj):
                    p = p + (pb[i, pl.ds(LANES * q, LANES)]
                             * vr[i, pl.ds(LANES * q, LANES)])
                pmat[prow_base + i, :] = p
                return carry

            lax.fori_loop(0, HCB, item_body, 0)

        def reduce_store(t16):
            sv = plsc.load_gather(pmat, [lanes, jnp.zeros((LANES,), jnp.int32)])
            for j in range(1, LANES):
                sv = sv + plsc.load_gather(
                    pmat, [lanes, jnp.full((LANES,), j, jnp.int32)])
            scores_v[pl.ds(t16 * LANES, LANES)] = sv

        fire_gather(0, 0)
        fire_gather(1, 1)
        fire_gather(2, 2)
        fire_gather(3, 3)

        def outer_body(k, carry):
            h = k * 4
            for j in range(4):
                hh = h + j
                s = j
                sm1 = (j + 3) % 4
                sm2 = (j + 2) % 4

                drain_gather(hh, s)

                @pl.when(hh >= 4)
                def _(_s=s):
                    drain_zero(_s)
                fire_pool(s)

                @pl.when(hh > 0)
                def _(_hh=hh, _s=sm1):
                    drain_pool(_s)
                    fire_copyback(_s)

                    @pl.when(_hh + 3 < n_half)
                    def _():
                        fire_gather(_hh + 3, _s)

                @pl.when(hh > 1)
                def _(_hh=hh, _s=sm2, _j=j):
                    drain_copyback(_s)
                    dot(_s, (_j % 2) * HCB)
                    fire_zero(_s)
                    if _j % 2 == 1:
                        reduce_store((_hh - 2) // 2)
            return carry

        lax.fori_loop(0, n_half // 4, outer_body, 0)

        s2, s3 = (n_half - 2) % 4, (n_half - 1) % 4
        drain_copyback(s2)
        dot(s2, ((n_half - 2) % 2) * HCB)
        drain_pool(s3)
        fire_copyback(s3)
        drain_copyback(s3)
        dot(s3, ((n_half - 1) % 2) * HCB)
        reduce_store((n_half - 1) // 2)

        pltpu.sync_copy(scores_v, out_hbm.at[pl.ds(wid * ipw, ipw)])

    return sc_scores


def _loss_body(s_ref, o_ref):
    s = s_ref[...]
    half = s.shape[0] // 2
    pos = s[:half, :]
    neg = s[half:, :]
    tot = jnp.sum(jax.nn.log_sigmoid(pos)) + jnp.sum(jax.nn.log_sigmoid(-neg))
    o_ref[...] = jnp.reshape(-tot, (1, 1))


def kernel(pos_u, pos_v, neg_u, neg_v, u_table, v_table):
    b, ctx = pos_u.shape
    d = u_table.shape[1]
    n_items = 2 * b
    assert n_items % NW == 0
    ipw = n_items // NW
    assert ipw % HCB == 0 and (ipw // HCB) % 4 == 0
    assert HCB * ctx == 2 * GATHER_ROWS
    assert GATHER_ROWS % ctx == 0 and 2 * (GATHER_ROWS // ctx) == HCB

    uidx = jnp.concatenate(
        [pos_u.reshape(-1), neg_u.reshape(-1)]).astype(jnp.int32)
    vidx = jnp.concatenate([pos_v, neg_v]).astype(jnp.int32)

    scores = _make_sc_scores(n_items, ctx, d, ipw)(
        uidx, vidx, u_table, v_table)

    scores2d = scores.reshape(n_items // 128, 128)
    loss = pl.pallas_call(
        _loss_body,
        out_shape=jax.ShapeDtypeStruct((1, 1), jnp.float32),
    )(scores2d)
    return loss[0, 0]


# speedup vs baseline: 1.1183x; 1.1183x over previous
import functools

import jax
import jax.numpy as jnp
from jax import lax
from jax.experimental import pallas as pl
from jax.experimental.pallas import tpu as pltpu
from jax.experimental.pallas import tpu_sc as plsc

NC = 2
NS = 16
LANES = 16
NW = NC * NS

HCB = 8
NSETS = 4
GATHER_ROWS = 80


def _make_sc_scores(n_items, ctx, d, ipw):
    n_half = ipw // HCB
    nj = d // LANES
    items_per_buf = GATHER_ROWS // ctx
    mesh = plsc.VectorSubcoreMesh(core_axis_name="c", subcore_axis_name="s")

    @functools.partial(
        pl.kernel,
        mesh=mesh,
        compiler_params=pltpu.CompilerParams(needs_layout_passes=False),
        out_type=jax.ShapeDtypeStruct((n_items,), jnp.float32),
        scratch_types=[
            pltpu.VMEM((ipw * ctx,), jnp.int32),
            pltpu.VMEM((ipw,), jnp.int32),
            pltpu.VMEM((GATHER_ROWS, d), jnp.float32),
            pltpu.VMEM((GATHER_ROWS, d), jnp.float32),
            pltpu.VMEM((GATHER_ROWS, d), jnp.float32),
            pltpu.VMEM((GATHER_ROWS, d), jnp.float32),
            pltpu.VMEM((GATHER_ROWS, d), jnp.float32),
            pltpu.VMEM((GATHER_ROWS, d), jnp.float32),
            pltpu.VMEM((GATHER_ROWS, d), jnp.float32),
            pltpu.VMEM((GATHER_ROWS, d), jnp.float32),
            pltpu.VMEM((HCB, d), jnp.float32),
            pltpu.VMEM((HCB, d), jnp.float32),
            pltpu.VMEM((HCB, d), jnp.float32),
            pltpu.VMEM((HCB, d), jnp.float32),
            pltpu.VMEM((HCB // 2, d), jnp.float32),
            pltpu.VMEM((HCB // 2, d), jnp.float32),
            pltpu.VMEM((HCB // 2, d), jnp.float32),
            pltpu.VMEM((HCB // 2, d), jnp.float32),
            pltpu.VMEM((HCB // 2, d), jnp.float32),
            pltpu.VMEM((GATHER_ROWS,), jnp.int32),
            pltpu.VMEM((ipw,), jnp.float32),
            pltpu.VMEM((LANES, LANES), jnp.float32),
            pltpu.VMEM_SHARED((NS * HCB, d), jnp.float32),
            pltpu.VMEM_SHARED((NS * HCB, d), jnp.float32),
            pltpu.VMEM_SHARED((NS * HCB, d), jnp.float32),
            pltpu.VMEM_SHARED((NS * HCB, d), jnp.float32),
            pltpu.SemaphoreType.DMA,
            pltpu.SemaphoreType.DMA,
            pltpu.SemaphoreType.DMA,
            pltpu.SemaphoreType.DMA,
            pltpu.SemaphoreType.DMA,
            pltpu.SemaphoreType.DMA,
            pltpu.SemaphoreType.DMA,
            pltpu.SemaphoreType.DMA,
            pltpu.SemaphoreType.DMA,
            pltpu.SemaphoreType.DMA,
            pltpu.SemaphoreType.DMA,
            pltpu.SemaphoreType.DMA,
        ],
    )
    def sc_scores(uidx_hbm, vidx_hbm, ut_hbm, vt_hbm, out_hbm,
                  uidx_v, vidx_v,
                  r00, r01, r10, r11, r20, r21, r30, r31,
                  vr0, vr1, vr2, vr3,
                  pb0, pb1, pb2, pb3, zbuf,
                  pidx0,
                  scores_v, pmat,
                  acc0, acc1, acc2, acc3,
                  gsem0, gsem1, gsem2, gsem3,
                  psem0, psem1, psem2, psem3,
                  csem0, csem1, csem2, csem3):
        wid = lax.axis_index("s") * NC + lax.axis_index("c")
        sid = lax.axis_index("s")
        rows_sets = ((r00, r01), (r10, r11), (r20, r21), (r30, r31))
        vr_sets = (vr0, vr1, vr2, vr3)
        pb_sets = (pb0, pb1, pb2, pb3)
        acc_sets = (acc0, acc1, acc2, acc3)
        gsems = (gsem0, gsem1, gsem2, gsem3)
        psems = (psem0, psem1, psem2, psem3)
        csems = (csem0, csem1, csem2, csem3)

        zeros16 = jnp.zeros((LANES,), jnp.float32)
        lanes = lax.iota(jnp.int32, LANES)
        my_rows = sid * HCB

        for k in range(GATHER_ROWS // LANES):
            r = k * LANES + lanes
            pidx0[pl.ds(k * LANES, LANES)] = my_rows + r // ctx

        def zb_body(i, c):
            for q in range(nj):
                zbuf[i, pl.ds(LANES * q, LANES)] = zeros16
            return c
        lax.fori_loop(0, HCB // 2, zb_body, 0)
        for a in acc_sets:
            pltpu.sync_copy(zbuf, a.at[pl.ds(my_rows, HCB // 2)])

        pltpu.sync_copy(uidx_hbm.at[pl.ds(wid * (ipw * ctx), ipw * ctx)], uidx_v)
        pltpu.sync_copy(vidx_hbm.at[pl.ds(wid * ipw, ipw)], vidx_v)

        def fire_gather(h, s):
            rows, vr, sem = rows_sets[s], vr_sets[s], gsems[s]
            bu = h * (HCB * ctx)
            for g in range(2):
                pltpu.make_async_copy(
                    ut_hbm.at[uidx_v.at[pl.ds(bu + g * GATHER_ROWS, GATHER_ROWS)]],
                    rows[g], sem).start()
            pltpu.make_async_copy(
                vt_hbm.at[vidx_v.at[pl.ds(h * HCB, HCB)]], vr, sem).start()

        def drain_gather(h, s):
            rows, vr, sem = rows_sets[s], vr_sets[s], gsems[s]
            bu = h * (HCB * ctx)
            for g in range(2):
                pltpu.make_async_copy(
                    ut_hbm.at[uidx_v.at[pl.ds(bu + g * GATHER_ROWS, GATHER_ROWS)]],
                    rows[g], sem).wait()
            pltpu.make_async_copy(
                vt_hbm.at[vidx_v.at[pl.ds(h * HCB, HCB)]], vr, sem).wait()

        def fire_pool(s):
            pltpu.async_copy(rows_sets[s][0], acc_sets[s].at[pidx0],
                             psems[s], add=True)

        def drain_pool(s):
            pltpu.make_async_copy(
                rows_sets[s][0], acc_sets[s].at[pidx0], psems[s]).wait()

        def fire_copyback(s):
            pltpu.make_async_copy(
                acc_sets[s].at[pl.ds(my_rows, HCB // 2)],
                pb_sets[s], csems[s]).start()

        def drain_copyback(s):
            pltpu.make_async_copy(
                acc_sets[s].at[pl.ds(my_rows, HCB // 2)],
                pb_sets[s], csems[s]).wait()

        def fire_zero(s):
            pltpu.make_async_copy(
                zbuf, acc_sets[s].at[pl.ds(my_rows, HCB // 2)],
                psems[s]).start()

        def drain_zero(s):
            pltpu.make_async_copy(
                zbuf, acc_sets[s].at[pl.ds(my_rows, HCB // 2)],
                psems[s]).wait()

        def dot(s, prow_base):
            pb, vr = pb_sets[s], vr_sets[s]

            def item_body(i, carry):
                p = pb[i, pl.ds(0, LANES)] * vr[i, pl.ds(0, LANES)]
                for q in range(1, nj):
                    p = p + (pb[i, pl.ds(LANES * q, LANES)]
                             * vr[i, pl.ds(LANES * q, LANES)])
                pmat[prow_base + i, :] = p
                return carry

            lax.fori_loop(0, HCB // 2, item_body, 0)

        def tec_pool_dot(s, prow_base):
            rows, vr = rows_sets[s][1], vr_sets[s]

            def item_body(i, carry):
                r0 = i * ctx
                a = [rows[r0, pl.ds(LANES * q, LANES)] for q in range(nj)]
                for c in range(1, ctx):
                    for q in range(nj):
                        a[q] = a[q] + rows[r0 + c, pl.ds(LANES * q, LANES)]
                vrow = items_per_buf + i
                p = a[0] * vr[vrow, pl.ds(0, LANES)]
                for q in range(1, nj):
                    p = p + a[q] * vr[vrow, pl.ds(LANES * q, LANES)]
                pmat[prow_base + items_per_buf + i, :] = p
                return carry

            lax.fori_loop(0, items_per_buf, item_body, 0)

        def reduce_store(t16):
            sv = plsc.load_gather(pmat, [lanes, jnp.zeros((LANES,), jnp.int32)])
            for j in range(1, LANES):
                sv = sv + plsc.load_gather(
                    pmat, [lanes, jnp.full((LANES,), j, jnp.int32)])
            scores_v[pl.ds(t16 * LANES, LANES)] = sv

        fire_gather(0, 0)
        fire_gather(1, 1)
        fire_gather(2, 2)
        fire_gather(3, 3)

        def outer_body(k, carry):
            h = k * 4
            for j in range(4):
                hh = h + j
                s = j
                sm1 = (j + 3) % 4
                sm2 = (j + 2) % 4

                drain_gather(hh, s)

                @pl.when(hh >= 4)
                def _(_s=s):
                    drain_zero(_s)
                fire_pool(s)

                @pl.when(hh > 0)
                def _(_hh=hh, _s=sm1, _j=j):
                    drain_pool(_s)
                    fire_copyback(_s)
                    tec_pool_dot(_s, ((_j + 1) % 2) * HCB)

                    @pl.when(_hh + 3 < n_half)
                    def _():
                        fire_gather(_hh + 3, _s)

                @pl.when(hh > 1)
                def _(_hh=hh, _s=sm2, _j=j):
                    drain_copyback(_s)
                    dot(_s, (_j % 2) * HCB)
                    fire_zero(_s)
                    if _j % 2 == 1:
                        reduce_store((_hh - 2) // 2)
            return carry

        lax.fori_loop(0, n_half // 4, outer_body, 0)

        s2, s3 = (n_half - 2) % 4, (n_half - 1) % 4
        drain_pool(s3)
        fire_copyback(s3)
        tec_pool_dot(s3, ((n_half - 1) % 2) * HCB)
        drain_copyback(s2)
        dot(s2, ((n_half - 2) % 2) * HCB)
        drain_copyback(s3)
        dot(s3, ((n_half - 1) % 2) * HCB)
        reduce_store((n_half - 1) // 2)

        pltpu.sync_copy(scores_v, out_hbm.at[pl.ds(wid * ipw, ipw)])

    return sc_scores


def _loss_body(s_ref, o_ref):
    s = s_ref[...]
    half = s.shape[0] // 2
    pos = s[:half, :]
    neg = s[half:, :]
    tot = jnp.sum(jax.nn.log_sigmoid(pos)) + jnp.sum(jax.nn.log_sigmoid(-neg))
    o_ref[...] = jnp.reshape(-tot, (1, 1))


def kernel(pos_u, pos_v, neg_u, neg_v, u_table, v_table):
    b, ctx = pos_u.shape
    d = u_table.shape[1]
    n_items = 2 * b
    assert n_items % NW == 0
    ipw = n_items // NW
    assert ipw % HCB == 0 and (ipw // HCB) % 4 == 0
    assert HCB * ctx == 2 * GATHER_ROWS
    assert GATHER_ROWS % ctx == 0 and 2 * (GATHER_ROWS // ctx) == HCB

    uidx = jnp.concatenate(
        [pos_u.reshape(-1), neg_u.reshape(-1)]).astype(jnp.int32)
    vidx = jnp.concatenate([pos_v, neg_v]).astype(jnp.int32)

    scores = _make_sc_scores(n_items, ctx, d, ipw)(
        uidx, vidx, u_table, v_table)

    scores2d = scores.reshape(n_items // 128, 128)
    loss = pl.pallas_call(
        _loss_body,
        out_shape=jax.ShapeDtypeStruct((1, 1), jnp.float32),
    )(scores2d)
    return loss[0, 0]


# speedup vs baseline: 1.1255x; 1.0064x over previous
import functools

import jax
import jax.numpy as jnp
from jax import lax
from jax.experimental import pallas as pl
from jax.experimental.pallas import tpu as pltpu
from jax.experimental.pallas import tpu_sc as plsc

NC = 2
NS = 16
LANES = 16
NW = NC * NS

HCB = 8
NSETS = 4
GATHER_ROWS = 80


def _make_sc_scores(n_items, ctx, d, ipw):
    n_half = ipw // HCB
    nj = d // LANES
    items_per_buf = GATHER_ROWS // ctx
    mesh = plsc.VectorSubcoreMesh(core_axis_name="c", subcore_axis_name="s")

    @functools.partial(
        pl.kernel,
        mesh=mesh,
        compiler_params=pltpu.CompilerParams(needs_layout_passes=False),
        out_type=jax.ShapeDtypeStruct((n_items,), jnp.float32),
        scratch_types=[
            pltpu.VMEM((ipw * ctx,), jnp.int32),
            pltpu.VMEM((ipw,), jnp.int32),
            pltpu.VMEM((GATHER_ROWS, d), jnp.float32),
            pltpu.VMEM((GATHER_ROWS, d), jnp.float32),
            pltpu.VMEM((GATHER_ROWS, d), jnp.float32),
            pltpu.VMEM((GATHER_ROWS, d), jnp.float32),
            pltpu.VMEM((GATHER_ROWS, d), jnp.float32),
            pltpu.VMEM((GATHER_ROWS, d), jnp.float32),
            pltpu.VMEM((GATHER_ROWS, d), jnp.float32),
            pltpu.VMEM((GATHER_ROWS, d), jnp.float32),
            pltpu.VMEM((HCB, d), jnp.float32),
            pltpu.VMEM((HCB, d), jnp.float32),
            pltpu.VMEM((HCB, d), jnp.float32),
            pltpu.VMEM((HCB, d), jnp.float32),
            pltpu.VMEM((HCB // 2, d), jnp.float32),
            pltpu.VMEM((HCB // 2, d), jnp.float32),
            pltpu.VMEM((HCB // 2, d), jnp.float32),
            pltpu.VMEM((HCB // 2, d), jnp.float32),
            pltpu.VMEM((HCB // 2, d), jnp.float32),
            pltpu.VMEM((GATHER_ROWS,), jnp.int32),
            pltpu.VMEM((ipw,), jnp.float32),
            pltpu.VMEM((LANES, LANES), jnp.float32),
            pltpu.VMEM_SHARED((NS * HCB, d), jnp.float32),
            pltpu.VMEM_SHARED((NS * HCB, d), jnp.float32),
            pltpu.VMEM_SHARED((NS * HCB, d), jnp.float32),
            pltpu.VMEM_SHARED((NS * HCB, d), jnp.float32),
            pltpu.SemaphoreType.DMA,
            pltpu.SemaphoreType.DMA,
            pltpu.SemaphoreType.DMA,
            pltpu.SemaphoreType.DMA,
            pltpu.SemaphoreType.DMA,
            pltpu.SemaphoreType.DMA,
            pltpu.SemaphoreType.DMA,
            pltpu.SemaphoreType.DMA,
            pltpu.SemaphoreType.DMA,
            pltpu.SemaphoreType.DMA,
            pltpu.SemaphoreType.DMA,
            pltpu.SemaphoreType.DMA,
        ],
    )
    def sc_scores(pu_hbm, pv_hbm, nu_hbm, nv_hbm, ut_hbm, vt_hbm, out_hbm,
                  uidx_v, vidx_v,
                  r00, r01, r10, r11, r20, r21, r30, r31,
                  vr0, vr1, vr2, vr3,
                  pb0, pb1, pb2, pb3, zbuf,
                  pidx0,
                  scores_v, pmat,
                  acc0, acc1, acc2, acc3,
                  gsem0, gsem1, gsem2, gsem3,
                  psem0, psem1, psem2, psem3,
                  csem0, csem1, csem2, csem3):
        wid = lax.axis_index("s") * NC + lax.axis_index("c")
        sid = lax.axis_index("s")
        rows_sets = ((r00, r01), (r10, r11), (r20, r21), (r30, r31))
        vr_sets = (vr0, vr1, vr2, vr3)
        pb_sets = (pb0, pb1, pb2, pb3)
        acc_sets = (acc0, acc1, acc2, acc3)
        gsems = (gsem0, gsem1, gsem2, gsem3)
        psems = (psem0, psem1, psem2, psem3)
        csems = (csem0, csem1, csem2, csem3)

        zeros16 = jnp.zeros((LANES,), jnp.float32)
        lanes = lax.iota(jnp.int32, LANES)
        my_rows = sid * HCB

        for k in range(GATHER_ROWS // LANES):
            r = k * LANES + lanes
            pidx0[pl.ds(k * LANES, LANES)] = my_rows + r // ctx

        def zb_body(i, c):
            for q in range(nj):
                zbuf[i, pl.ds(LANES * q, LANES)] = zeros16
            return c
        lax.fori_loop(0, HCB // 2, zb_body, 0)
        for a in acc_sets:
            pltpu.sync_copy(zbuf, a.at[pl.ds(my_rows, HCB // 2)])

        half_w = NW // 2

        @pl.when(wid < half_w)
        def _():
            pltpu.sync_copy(pu_hbm.at[pl.ds(wid * (ipw * ctx), ipw * ctx)],
                            uidx_v)
            pltpu.sync_copy(pv_hbm.at[pl.ds(wid * ipw, ipw)], vidx_v)

        @pl.when(wid >= half_w)
        def _():
            pltpu.sync_copy(
                nu_hbm.at[pl.ds((wid - half_w) * (ipw * ctx), ipw * ctx)],
                uidx_v)
            pltpu.sync_copy(nv_hbm.at[pl.ds((wid - half_w) * ipw, ipw)], vidx_v)

        def fire_gather(h, s):
            rows, vr, sem = rows_sets[s], vr_sets[s], gsems[s]
            bu = h * (HCB * ctx)
            for g in range(2):
                pltpu.make_async_copy(
                    ut_hbm.at[uidx_v.at[pl.ds(bu + g * GATHER_ROWS, GATHER_ROWS)]],
                    rows[g], sem).start()
            pltpu.make_async_copy(
                vt_hbm.at[vidx_v.at[pl.ds(h * HCB, HCB)]], vr, sem).start()

        def drain_gather(h, s):
            rows, vr, sem = rows_sets[s], vr_sets[s], gsems[s]
            bu = h * (HCB * ctx)
            for g in range(2):
                pltpu.make_async_copy(
                    ut_hbm.at[uidx_v.at[pl.ds(bu + g * GATHER_ROWS, GATHER_ROWS)]],
                    rows[g], sem).wait()
            pltpu.make_async_copy(
                vt_hbm.at[vidx_v.at[pl.ds(h * HCB, HCB)]], vr, sem).wait()

        def fire_pool(s):
            pltpu.async_copy(rows_sets[s][0], acc_sets[s].at[pidx0],
                             psems[s], add=True)

        def drain_pool(s):
            pltpu.make_async_copy(
                rows_sets[s][0], acc_sets[s].at[pidx0], psems[s]).wait()

        def fire_copyback(s):
            pltpu.make_async_copy(
                acc_sets[s].at[pl.ds(my_rows, HCB // 2)],
                pb_sets[s], csems[s]).start()

        def drain_copyback(s):
            pltpu.make_async_copy(
                acc_sets[s].at[pl.ds(my_rows, HCB // 2)],
                pb_sets[s], csems[s]).wait()

        def fire_zero(s):
            pltpu.make_async_copy(
                zbuf, acc_sets[s].at[pl.ds(my_rows, HCB // 2)],
                psems[s]).start()

        def drain_zero(s):
            pltpu.make_async_copy(
                zbuf, acc_sets[s].at[pl.ds(my_rows, HCB // 2)],
                psems[s]).wait()

        def dot(s, prow_base):
            pb, vr = pb_sets[s], vr_sets[s]

            def item_body(i, carry):
                p = pb[i, pl.ds(0, LANES)] * vr[i, pl.ds(0, LANES)]
                for q in range(1, nj):
                    p = p + (pb[i, pl.ds(LANES * q, LANES)]
                             * vr[i, pl.ds(LANES * q, LANES)])
                pmat[prow_base + i, :] = p
                return carry

            lax.fori_loop(0, HCB // 2, item_body, 0)

        def tec_pool_dot(s, prow_base):
            rows, vr = rows_sets[s][1], vr_sets[s]

            def item_body(i, carry):
                r0 = i * ctx
                a = [rows[r0, pl.ds(LANES * q, LANES)] for q in range(nj)]
                for c in range(1, ctx):
                    for q in range(nj):
                        a[q] = a[q] + rows[r0 + c, pl.ds(LANES * q, LANES)]
                vrow = items_per_buf + i
                p = a[0] * vr[vrow, pl.ds(0, LANES)]
                for q in range(1, nj):
                    p = p + a[q] * vr[vrow, pl.ds(LANES * q, LANES)]
                pmat[prow_base + items_per_buf + i, :] = p
                return carry

            lax.fori_loop(0, items_per_buf, item_body, 0)

        def reduce_store(t16):
            sv = plsc.load_gather(pmat, [lanes, jnp.zeros((LANES,), jnp.int32)])
            for j in range(1, LANES):
                sv = sv + plsc.load_gather(
                    pmat, [lanes, jnp.full((LANES,), j, jnp.int32)])
            scores_v[pl.ds(t16 * LANES, LANES)] = sv

        fire_gather(0, 0)
        fire_gather(1, 1)
        fire_gather(2, 2)
        fire_gather(3, 3)

        def outer_body(k, carry):
            h = k * 4
            for j in range(4):
                hh = h + j
                s = j
                sm1 = (j + 3) % 4
                sm2 = (j + 2) % 4

                drain_gather(hh, s)

                @pl.when(hh >= 4)
                def _(_s=s):
                    drain_zero(_s)
                fire_pool(s)

                @pl.when(hh > 0)
                def _(_hh=hh, _s=sm1, _j=j):
                    drain_pool(_s)
                    fire_copyback(_s)
                    tec_pool_dot(_s, ((_j + 1) % 2) * HCB)

                    @pl.when(_hh + 3 < n_half)
                    def _():
                        fire_gather(_hh + 3, _s)

                @pl.when(hh > 1)
                def _(_hh=hh, _s=sm2, _j=j):
                    drain_copyback(_s)
                    dot(_s, (_j % 2) * HCB)
                    fire_zero(_s)
                    if _j % 2 == 1:
                        reduce_store((_hh - 2) // 2)
            return carry

        lax.fori_loop(0, n_half // 4, outer_body, 0)

        s2, s3 = (n_half - 2) % 4, (n_half - 1) % 4
        drain_pool(s3)
        fire_copyback(s3)
        tec_pool_dot(s3, ((n_half - 1) % 2) * HCB)
        drain_copyback(s2)
        dot(s2, ((n_half - 2) % 2) * HCB)
        drain_copyback(s3)
        dot(s3, ((n_half - 1) % 2) * HCB)
        reduce_store((n_half - 1) // 2)

        pltpu.sync_copy(scores_v, out_hbm.at[pl.ds(wid * ipw, ipw)])

    return sc_scores


def _loss_body(s_ref, o_ref):
    s = s_ref[...]
    half = s.shape[0] // 2
    pos = s[:half, :]
    neg = s[half:, :]
    tot = jnp.sum(jax.nn.log_sigmoid(pos)) + jnp.sum(jax.nn.log_sigmoid(-neg))
    o_ref[...] = jnp.reshape(-tot, (1, 1))


def kernel(pos_u, pos_v, neg_u, neg_v, u_table, v_table):
    b, ctx = pos_u.shape
    d = u_table.shape[1]
    n_items = 2 * b
    assert n_items % NW == 0
    ipw = n_items // NW
    assert ipw % HCB == 0 and (ipw // HCB) % 4 == 0
    assert HCB * ctx == 2 * GATHER_ROWS
    assert GATHER_ROWS % ctx == 0 and 2 * (GATHER_ROWS // ctx) == HCB

    scores = _make_sc_scores(n_items, ctx, d, ipw)(
        pos_u.reshape(-1).astype(jnp.int32), pos_v.astype(jnp.int32),
        neg_u.reshape(-1).astype(jnp.int32), neg_v.astype(jnp.int32),
        u_table, v_table)

    scores2d = scores.reshape(n_items // 128, 128)
    loss = pl.pallas_call(
        _loss_body,
        out_shape=jax.ShapeDtypeStruct((1, 1), jnp.float32),
    )(scores2d)
    return loss[0, 0]


# speedup vs baseline: 1.1677x; 1.0375x over previous
import functools

import jax
import jax.numpy as jnp
from jax import lax
from jax.experimental import pallas as pl
from jax.experimental.pallas import tpu as pltpu
from jax.experimental.pallas import tpu_sc as plsc

NC = 2
NS = 16
LANES = 16
NW = NC * NS

HCB = 8
NSETS = 4
SITEMS = 2
TITEMS = HCB - SITEMS
GA = SITEMS * 20
GB = TITEMS * 20


def _make_sc_scores(n_items, ctx, d, ipw):
    n_half = ipw // HCB
    nj = d // LANES
    ga, gb = SITEMS * ctx, TITEMS * ctx
    mesh = plsc.VectorSubcoreMesh(core_axis_name="c", subcore_axis_name="s")

    @functools.partial(
        pl.kernel,
        mesh=mesh,
        compiler_params=pltpu.CompilerParams(needs_layout_passes=False),
        out_type=jax.ShapeDtypeStruct((n_items,), jnp.float32),
        scratch_types=[
            pltpu.VMEM((ipw * ctx,), jnp.int32),
            pltpu.VMEM((ipw,), jnp.int32),
            pltpu.VMEM((ga, d), jnp.float32),
            pltpu.VMEM((gb, d), jnp.float32),
            pltpu.VMEM((ga, d), jnp.float32),
            pltpu.VMEM((gb, d), jnp.float32),
            pltpu.VMEM((ga, d), jnp.float32),
            pltpu.VMEM((gb, d), jnp.float32),
            pltpu.VMEM((ga, d), jnp.float32),
            pltpu.VMEM((gb, d), jnp.float32),
            pltpu.VMEM((HCB, d), jnp.float32),
            pltpu.VMEM((HCB, d), jnp.float32),
            pltpu.VMEM((HCB, d), jnp.float32),
            pltpu.VMEM((HCB, d), jnp.float32),
            pltpu.VMEM((SITEMS, d), jnp.float32),
            pltpu.VMEM((SITEMS, d), jnp.float32),
            pltpu.VMEM((SITEMS, d), jnp.float32),
            pltpu.VMEM((SITEMS, d), jnp.float32),
            pltpu.VMEM((SITEMS, d), jnp.float32),
            pltpu.VMEM((ga,), jnp.int32),
            pltpu.VMEM((ipw,), jnp.float32),
            pltpu.VMEM((LANES, LANES), jnp.float32),
            pltpu.VMEM_SHARED((NS * HCB, d), jnp.float32),
            pltpu.VMEM_SHARED((NS * HCB, d), jnp.float32),
            pltpu.VMEM_SHARED((NS * HCB, d), jnp.float32),
            pltpu.VMEM_SHARED((NS * HCB, d), jnp.float32),
            pltpu.SemaphoreType.DMA,
            pltpu.SemaphoreType.DMA,
            pltpu.SemaphoreType.DMA,
            pltpu.SemaphoreType.DMA,
            pltpu.SemaphoreType.DMA,
            pltpu.SemaphoreType.DMA,
            pltpu.SemaphoreType.DMA,
            pltpu.SemaphoreType.DMA,
            pltpu.SemaphoreType.DMA,
            pltpu.SemaphoreType.DMA,
            pltpu.SemaphoreType.DMA,
            pltpu.SemaphoreType.DMA,
        ],
    )
    def sc_scores(pu_hbm, pv_hbm, nu_hbm, nv_hbm, ut_hbm, vt_hbm, out_hbm,
                  uidx_v, vidx_v,
                  r00, r01, r10, r11, r20, r21, r30, r31,
                  vr0, vr1, vr2, vr3,
                  pb0, pb1, pb2, pb3, zbuf,
                  pidx0,
                  scores_v, pmat,
                  acc0, acc1, acc2, acc3,
                  gsem0, gsem1, gsem2, gsem3,
                  psem0, psem1, psem2, psem3,
                  csem0, csem1, csem2, csem3):
        wid = lax.axis_index("s") * NC + lax.axis_index("c")
        sid = lax.axis_index("s")
        rows_sets = ((r00, r01), (r10, r11), (r20, r21), (r30, r31))
        vr_sets = (vr0, vr1, vr2, vr3)
        pb_sets = (pb0, pb1, pb2, pb3)
        acc_sets = (acc0, acc1, acc2, acc3)
        gsems = (gsem0, gsem1, gsem2, gsem3)
        psems = (psem0, psem1, psem2, psem3)
        csems = (csem0, csem1, csem2, csem3)

        zeros16 = jnp.zeros((LANES,), jnp.float32)
        lanes = lax.iota(jnp.int32, LANES)
        my_rows = sid * HCB

        starts = list(range(0, ga - LANES + 1, LANES))
        if starts[-1] != ga - LANES:
            starts.append(ga - LANES)
        for st in starts:
            r = st + lanes
            pidx0[pl.ds(st, LANES)] = my_rows + r // ctx

        def zb_body(i, c):
            for q in range(nj):
                zbuf[i, pl.ds(LANES * q, LANES)] = zeros16
            return c
        lax.fori_loop(0, SITEMS, zb_body, 0)
        for a in acc_sets:
            pltpu.sync_copy(zbuf, a.at[pl.ds(my_rows, SITEMS)])

        half_w = NW // 2

        @pl.when(wid < half_w)
        def _():
            pltpu.sync_copy(pu_hbm.at[pl.ds(wid * (ipw * ctx), ipw * ctx)],
                            uidx_v)
            pltpu.sync_copy(pv_hbm.at[pl.ds(wid * ipw, ipw)], vidx_v)

        @pl.when(wid >= half_w)
        def _():
            pltpu.sync_copy(
                nu_hbm.at[pl.ds((wid - half_w) * (ipw * ctx), ipw * ctx)],
                uidx_v)
            pltpu.sync_copy(nv_hbm.at[pl.ds((wid - half_w) * ipw, ipw)], vidx_v)

        def fire_gather(h, s):
            rows, vr, sem = rows_sets[s], vr_sets[s], gsems[s]
            bu = h * (HCB * ctx)
            pltpu.make_async_copy(
                ut_hbm.at[uidx_v.at[pl.ds(bu, ga)]], rows[0], sem).start()
            pltpu.make_async_copy(
                ut_hbm.at[uidx_v.at[pl.ds(bu + ga, gb)]], rows[1], sem).start()
            pltpu.make_async_copy(
                vt_hbm.at[vidx_v.at[pl.ds(h * HCB, HCB)]], vr, sem).start()

        def drain_gather(h, s):
            rows, vr, sem = rows_sets[s], vr_sets[s], gsems[s]
            bu = h * (HCB * ctx)
            pltpu.make_async_copy(
                ut_hbm.at[uidx_v.at[pl.ds(bu, ga)]], rows[0], sem).wait()
            pltpu.make_async_copy(
                ut_hbm.at[uidx_v.at[pl.ds(bu + ga, gb)]], rows[1], sem).wait()
            pltpu.make_async_copy(
                vt_hbm.at[vidx_v.at[pl.ds(h * HCB, HCB)]], vr, sem).wait()

        def fire_pool(s):
            pltpu.async_copy(rows_sets[s][0], acc_sets[s].at[pidx0],
                             psems[s], add=True)

        def drain_pool(s):
            pltpu.make_async_copy(
                rows_sets[s][0], acc_sets[s].at[pidx0], psems[s]).wait()

        def fire_copyback(s):
            pltpu.make_async_copy(
                acc_sets[s].at[pl.ds(my_rows, SITEMS)],
                pb_sets[s], csems[s]).start()

        def drain_copyback(s):
            pltpu.make_async_copy(
                acc_sets[s].at[pl.ds(my_rows, SITEMS)],
                pb_sets[s], csems[s]).wait()

        def fire_zero(s):
            pltpu.make_async_copy(
                zbuf, acc_sets[s].at[pl.ds(my_rows, SITEMS)],
                psems[s]).start()

        def drain_zero(s):
            pltpu.make_async_copy(
                zbuf, acc_sets[s].at[pl.ds(my_rows, SITEMS)],
                psems[s]).wait()

        def dot(s, prow_base):
            pb, vr = pb_sets[s], vr_sets[s]

            def item_body(i, carry):
                p = pb[i, pl.ds(0, LANES)] * vr[i, pl.ds(0, LANES)]
                for q in range(1, nj):
                    p = p + (pb[i, pl.ds(LANES * q, LANES)]
                             * vr[i, pl.ds(LANES * q, LANES)])
                pmat[prow_base + i, :] = p
                return carry

            lax.fori_loop(0, SITEMS, item_body, 0)

        def tec_pool_dot(s, prow_base):
            rows, vr = rows_sets[s][1], vr_sets[s]

            def item_body(i, carry):
                r0 = i * ctx
                a = [rows[r0, pl.ds(LANES * q, LANES)] for q in range(nj)]
                for c in range(1, ctx):
                    for q in range(nj):
                        a[q] = a[q] + rows[r0 + c, pl.ds(LANES * q, LANES)]
                vrow = SITEMS + i
                p = a[0] * vr[vrow, pl.ds(0, LANES)]
                for q in range(1, nj):
                    p = p + a[q] * vr[vrow, pl.ds(LANES * q, LANES)]
                pmat[prow_base + SITEMS + i, :] = p
                return carry

            lax.fori_loop(0, TITEMS, item_body, 0)

        def reduce_store(t16):
            sv = plsc.load_gather(pmat, [lanes, jnp.zeros((LANES,), jnp.int32)])
            for j in range(1, LANES):
                sv = sv + plsc.load_gather(
                    pmat, [lanes, jnp.full((LANES,), j, jnp.int32)])
            scores_v[pl.ds(t16 * LANES, LANES)] = sv

        fire_gather(0, 0)
        fire_gather(1, 1)
        fire_gather(2, 2)
        fire_gather(3, 3)

        def outer_body(k, carry):
            h = k * 4
            for j in range(4):
                hh = h + j
                s = j
                sm1 = (j + 3) % 4
                sm2 = (j + 2) % 4

                drain_gather(hh, s)

                @pl.when(hh >= 4)
                def _(_s=s):
                    drain_zero(_s)
                fire_pool(s)

                @pl.when(hh > 0)
                def _(_hh=hh, _s=sm1, _j=j):
                    drain_pool(_s)
                    fire_copyback(_s)
                    tec_pool_dot(_s, ((_j + 1) % 2) * HCB)

                    @pl.when(_hh + 3 < n_half)
                    def _():
                        fire_gather(_hh + 3, _s)

                @pl.when(hh > 1)
                def _(_hh=hh, _s=sm2, _j=j):
                    drain_copyback(_s)
                    dot(_s, (_j % 2) * HCB)
                    fire_zero(_s)
                    if _j % 2 == 1:
                        reduce_store((_hh - 2) // 2)
            return carry

        lax.fori_loop(0, n_half // 4, outer_body, 0)

        s2, s3 = (n_half - 2) % 4, (n_half - 1) % 4
        drain_pool(s3)
        fire_copyback(s3)
        tec_pool_dot(s3, ((n_half - 1) % 2) * HCB)
        drain_copyback(s2)
        dot(s2, ((n_half - 2) % 2) * HCB)
        drain_copyback(s3)
        dot(s3, ((n_half - 1) % 2) * HCB)
        reduce_store((n_half - 1) // 2)

        pltpu.sync_copy(scores_v, out_hbm.at[pl.ds(wid * ipw, ipw)])

    return sc_scores


def _loss_body(s_ref, o_ref):
    s = s_ref[...]
    half = s.shape[0] // 2
    pos = s[:half, :]
    neg = s[half:, :]
    tot = jnp.sum(jax.nn.log_sigmoid(pos)) + jnp.sum(jax.nn.log_sigmoid(-neg))
    o_ref[...] = jnp.reshape(-tot, (1, 1))


def kernel(pos_u, pos_v, neg_u, neg_v, u_table, v_table):
    b, ctx = pos_u.shape
    d = u_table.shape[1]
    n_items = 2 * b
    assert n_items % NW == 0
    ipw = n_items // NW
    assert ipw % HCB == 0 and (ipw // HCB) % 4 == 0
    assert SITEMS * ctx <= 128 and TITEMS * ctx <= 128
    assert (SITEMS * ctx) % 8 == 0

    scores = _make_sc_scores(n_items, ctx, d, ipw)(
        pos_u.reshape(-1).astype(jnp.int32), pos_v.astype(jnp.int32),
        neg_u.reshape(-1).astype(jnp.int32), neg_v.astype(jnp.int32),
        u_table, v_table)

    scores2d = scores.reshape(n_items // 128, 128)
    loss = pl.pallas_call(
        _loss_body,
        out_shape=jax.ShapeDtypeStruct((1, 1), jnp.float32),
    )(scores2d)
    return loss[0, 0]


# speedup vs baseline: 1.1683x; 1.0006x over previous
import functools

import jax
import jax.numpy as jnp
from jax import lax
from jax.experimental import pallas as pl
from jax.experimental.pallas import tpu as pltpu
from jax.experimental.pallas import tpu_sc as plsc

NC = 2
NS = 16
LANES = 16
NW = NC * NS

HCB = 8
NSETS = 4
SITEMS = 2
TITEMS = HCB - SITEMS


def _make_sc_scores(n_items, ctx, d, ipw):
    n_half = ipw // HCB
    nj = d // LANES
    ga, gb = SITEMS * ctx, TITEMS * ctx
    mesh = plsc.VectorSubcoreMesh(core_axis_name="c", subcore_axis_name="s")

    @functools.partial(
        pl.kernel,
        mesh=mesh,
        compiler_params=pltpu.CompilerParams(needs_layout_passes=False),
        out_type=jax.ShapeDtypeStruct((n_items,), jnp.float32),
        scratch_types=[
            pltpu.VMEM((ipw * ctx,), jnp.int32),
            pltpu.VMEM((ipw,), jnp.int32),
            pltpu.VMEM((ga, d), jnp.float32),
            pltpu.VMEM((gb, d), jnp.float32),
            pltpu.VMEM((ga, d), jnp.float32),
            pltpu.VMEM((gb, d), jnp.float32),
            pltpu.VMEM((ga, d), jnp.float32),
            pltpu.VMEM((gb, d), jnp.float32),
            pltpu.VMEM((ga, d), jnp.float32),
            pltpu.VMEM((gb, d), jnp.float32),
            pltpu.VMEM((HCB, d), jnp.float32),
            pltpu.VMEM((HCB, d), jnp.float32),
            pltpu.VMEM((HCB, d), jnp.float32),
            pltpu.VMEM((HCB, d), jnp.float32),
            pltpu.VMEM((SITEMS, d), jnp.float32),
            pltpu.VMEM((SITEMS, d), jnp.float32),
            pltpu.VMEM((SITEMS, d), jnp.float32),
            pltpu.VMEM((SITEMS, d), jnp.float32),
            pltpu.VMEM((SITEMS, d), jnp.float32),
            pltpu.VMEM((ga,), jnp.int32),
            pltpu.VMEM((ipw,), jnp.float32),
            pltpu.VMEM((LANES, LANES), jnp.float32),
            pltpu.VMEM_SHARED((NS * HCB, d), jnp.float32),
            pltpu.VMEM_SHARED((NS * HCB, d), jnp.float32),
            pltpu.VMEM_SHARED((NS * HCB, d), jnp.float32),
            pltpu.VMEM_SHARED((NS * HCB, d), jnp.float32),
            pltpu.SemaphoreType.DMA,
            pltpu.SemaphoreType.DMA,
            pltpu.SemaphoreType.DMA,
            pltpu.SemaphoreType.DMA,
            pltpu.SemaphoreType.DMA,
            pltpu.SemaphoreType.DMA,
            pltpu.SemaphoreType.DMA,
            pltpu.SemaphoreType.DMA,
            pltpu.SemaphoreType.DMA,
            pltpu.SemaphoreType.DMA,
            pltpu.SemaphoreType.DMA,
            pltpu.SemaphoreType.DMA,
        ],
    )
    def sc_scores(pu_hbm, pv_hbm, nu_hbm, nv_hbm, ut_hbm, vt_hbm, out_hbm,
                  uidx_v, vidx_v,
                  r00, r01, r10, r11, r20, r21, r30, r31,
                  vr0, vr1, vr2, vr3,
                  pb0, pb1, pb2, pb3, zbuf,
                  pidx0,
                  scores_v, pmat,
                  acc0, acc1, acc2, acc3,
                  gsem0, gsem1, gsem2, gsem3,
                  psem0, psem1, psem2, psem3,
                  csem0, csem1, csem2, csem3):
        wid = lax.axis_index("s") * NC + lax.axis_index("c")
        sid = lax.axis_index("s")
        rows_sets = ((r00, r01), (r10, r11), (r20, r21), (r30, r31))
        vr_sets = (vr0, vr1, vr2, vr3)
        pb_sets = (pb0, pb1, pb2, pb3)
        acc_sets = (acc0, acc1, acc2, acc3)
        gsems = (gsem0, gsem1, gsem2, gsem3)
        psems = (psem0, psem1, psem2, psem3)
        csems = (csem0, csem1, csem2, csem3)

        zeros16 = jnp.zeros((LANES,), jnp.float32)
        lanes = lax.iota(jnp.int32, LANES)
        my_rows = sid * HCB

        starts = list(range(0, ga - LANES + 1, LANES))
        if starts[-1] != ga - LANES:
            starts.append(ga - LANES)
        for st in starts:
            r = st + lanes
            pidx0[pl.ds(st, LANES)] = my_rows + r // ctx

        def zb_body(i, c):
            for q in range(nj):
                zbuf[i, pl.ds(LANES * q, LANES)] = zeros16
            return c
        lax.fori_loop(0, SITEMS, zb_body, 0)
        for a in acc_sets:
            pltpu.sync_copy(zbuf, a.at[pl.ds(my_rows, SITEMS)])

        half_w = NW // 2

        @pl.when(wid < half_w)
        def _():
            pltpu.sync_copy(pu_hbm.at[pl.ds(wid * (ipw * ctx), ipw * ctx)],
                            uidx_v)
            pltpu.sync_copy(pv_hbm.at[pl.ds(wid * ipw, ipw)], vidx_v)

        @pl.when(wid >= half_w)
        def _():
            pltpu.sync_copy(
                nu_hbm.at[pl.ds((wid - half_w) * (ipw * ctx), ipw * ctx)],
                uidx_v)
            pltpu.sync_copy(nv_hbm.at[pl.ds((wid - half_w) * ipw, ipw)], vidx_v)

        def fire_gather(h, s):
            rows, vr, sem = rows_sets[s], vr_sets[s], gsems[s]
            bu = h * (HCB * ctx)
            pltpu.make_async_copy(
                ut_hbm.at[uidx_v.at[pl.ds(bu, ga)]], rows[0], sem).start()
            pltpu.make_async_copy(
                ut_hbm.at[uidx_v.at[pl.ds(bu + ga, gb)]], rows[1], sem).start()
            pltpu.make_async_copy(
                vt_hbm.at[vidx_v.at[pl.ds(h * HCB, HCB)]], vr, sem).start()

        def drain_gather(h, s):
            rows, vr, sem = rows_sets[s], vr_sets[s], gsems[s]
            bu = h * (HCB * ctx)
            pltpu.make_async_copy(
                ut_hbm.at[uidx_v.at[pl.ds(bu, ga)]], rows[0], sem).wait()
            pltpu.make_async_copy(
                ut_hbm.at[uidx_v.at[pl.ds(bu + ga, gb)]], rows[1], sem).wait()
            pltpu.make_async_copy(
                vt_hbm.at[vidx_v.at[pl.ds(h * HCB, HCB)]], vr, sem).wait()

        def fire_pool(s):
            pltpu.async_copy(rows_sets[s][0], acc_sets[s].at[pidx0],
                             psems[s], add=True)

        def drain_pool(s):
            pltpu.make_async_copy(
                rows_sets[s][0], acc_sets[s].at[pidx0], psems[s]).wait()

        def fire_copyback(s):
            pltpu.make_async_copy(
                acc_sets[s].at[pl.ds(my_rows, SITEMS)],
                pb_sets[s], csems[s]).start()

        def drain_copyback(s):
            pltpu.make_async_copy(
                acc_sets[s].at[pl.ds(my_rows, SITEMS)],
                pb_sets[s], csems[s]).wait()

        def fire_zero(s):
            pltpu.make_async_copy(
                zbuf, acc_sets[s].at[pl.ds(my_rows, SITEMS)],
                psems[s]).start()

        def drain_zero(s):
            pltpu.make_async_copy(
                zbuf, acc_sets[s].at[pl.ds(my_rows, SITEMS)],
                psems[s]).wait()

        def dot(s, prow_base):
            pb, vr = pb_sets[s], vr_sets[s]

            def item_body(i, carry):
                p = pb[i, pl.ds(0, LANES)] * vr[i, pl.ds(0, LANES)]
                for q in range(1, nj):
                    p = p + (pb[i, pl.ds(LANES * q, LANES)]
                             * vr[i, pl.ds(LANES * q, LANES)])
                pmat[prow_base + i, :] = p
                return carry

            lax.fori_loop(0, SITEMS, item_body, 0)

        def tec_pool_dot(s, prow_base):
            rows, vr = rows_sets[s][1], vr_sets[s]

            def item_body(i, carry):
                r0 = i * ctx
                a = [rows[r0, pl.ds(LANES * q, LANES)] for q in range(nj)]
                for c in range(1, ctx):
                    for q in range(nj):
                        a[q] = a[q] + rows[r0 + c, pl.ds(LANES * q, LANES)]
                vrow = SITEMS + i
                p = a[0] * vr[vrow, pl.ds(0, LANES)]
                for q in range(1, nj):
                    p = p + a[q] * vr[vrow, pl.ds(LANES * q, LANES)]
                pmat[prow_base + SITEMS + i, :] = p
                return carry

            lax.fori_loop(0, TITEMS, item_body, 0)

        def reduce_store(t16):
            sv = plsc.load_gather(pmat, [lanes, jnp.zeros((LANES,), jnp.int32)])
            for j in range(1, LANES):
                sv = sv + plsc.load_gather(
                    pmat, [lanes, jnp.full((LANES,), j, jnp.int32)])
            scores_v[pl.ds(t16 * LANES, LANES)] = sv

        fire_gather(0, 0)
        fire_gather(1, 1)
        fire_gather(2, 2)
        fire_gather(3, 3)

        def outer_body(k, carry):
            h = k * 4
            for j in range(4):
                hh = h + j
                s = j
                sm1 = (j + 3) % 4
                sm2 = (j + 2) % 4

                drain_gather(hh, s)

                @pl.when(hh >= 4)
                def _(_s=s):
                    drain_zero(_s)
                fire_pool(s)

                @pl.when(hh > 0)
                def _(_hh=hh, _s=sm1, _j=j):
                    drain_pool(_s)
                    fire_copyback(_s)
                    tec_pool_dot(_s, ((_j + 1) % 2) * HCB)

                    @pl.when(_hh + 3 < n_half)
                    def _():
                        fire_gather(_hh + 3, _s)

                @pl.when(hh > 1)
                def _(_hh=hh, _s=sm2, _j=j):
                    drain_copyback(_s)
                    dot(_s, (_j % 2) * HCB)
                    fire_zero(_s)
                    if _j % 2 == 1:
                        reduce_store((_hh - 2) // 2)
            return carry

        lax.fori_loop(0, n_half // 4, outer_body, 0)

        s2, s3 = (n_half - 2) % 4, (n_half - 1) % 4
        drain_pool(s3)
        fire_copyback(s3)
        tec_pool_dot(s3, ((n_half - 1) % 2) * HCB)
        drain_copyback(s2)
        dot(s2, ((n_half - 2) % 2) * HCB)
        drain_copyback(s3)
        dot(s3, ((n_half - 1) % 2) * HCB)
        reduce_store((n_half - 1) // 2)

        pltpu.sync_copy(scores_v, out_hbm.at[pl.ds(wid * ipw, ipw)])

    return sc_scores


def _loss_body(s_ref, o_ref):
    s = s_ref[...]
    half = s.shape[0] // 2
    pos = s[:half, :]
    neg = s[half:, :]
    tot = jnp.sum(jax.nn.log_sigmoid(pos)) + jnp.sum(jax.nn.log_sigmoid(-neg))
    o_ref[...] = jnp.reshape(-tot, (1, 1))


def kernel(pos_u, pos_v, neg_u, neg_v, u_table, v_table):
    b, ctx = pos_u.shape
    d = u_table.shape[1]
    n_items = 2 * b
    assert n_items % NW == 0
    ipw = n_items // NW
    assert ipw % HCB == 0 and (ipw // HCB) % 4 == 0
    assert SITEMS * ctx <= 128 and TITEMS * ctx <= 128
    assert (SITEMS * ctx) % 8 == 0

    scores = _make_sc_scores(n_items, ctx, d, ipw)(
        pos_u.reshape(-1).astype(jnp.int32), pos_v.astype(jnp.int32),
        neg_u.reshape(-1).astype(jnp.int32), neg_v.astype(jnp.int32),
        u_table, v_table)

    scores2d = scores.reshape(n_items // 128, 128)
    loss = pl.pallas_call(
        _loss_body,
        out_shape=jax.ShapeDtypeStruct((1, 1), jnp.float32),
    )(scores2d)
    return loss[0, 0]
